# baseline (speedup 1.0000x reference)
import functools
import math

import jax
import jax.numpy as jnp
from jax.experimental import pallas as pl
from jax.experimental.pallas import tpu as pltpu

N = 10000
NFEAT = 128
NHID = 64
NCLASS = 32
EPS = 1e-5

MB = 400
KB = 2048
NM = N // MB
NK = -(-N // KB)


def _zcat_body(x_ref, w_ref, o_ref):
    o_ref[...] = jnp.dot(x_ref[...], w_ref[...],
                         preferred_element_type=jnp.float32)


def _layer2_body(adj_ref, adj1_ref, adj2_ref, z_ref, b_ref, fw_ref, fb_ref,
                 bng_ref, bnb_ref, mw_ref, mb_ref, o_ref, acc_ref, *, nk):
    k = pl.program_id(1)

    @pl.when(k == 0)
    def _init():
        acc_ref[...] = jnp.zeros_like(acc_ref)

    z = z_ref[...]
    valid = jax.lax.broadcasted_iota(jnp.int32, z.shape, 0) < (N - k * KB)
    z = jnp.where(valid, z, 0.0)
    p0 = jnp.dot(adj_ref[...], z[:, 0:NHID],
                 preferred_element_type=jnp.float32)
    p1 = jnp.dot(adj1_ref[...], z[:, NHID:2 * NHID],
                 preferred_element_type=jnp.float32)
    p2 = jnp.dot(adj2_ref[...], z[:, 2 * NHID:3 * NHID],
                 preferred_element_type=jnp.float32)
    acc_ref[...] += jnp.concatenate([p0, p1, p2], axis=1)

    @pl.when(k == nk - 1)
    def _epilogue():
        xcat = jnp.maximum(acc_ref[...] + b_ref[...], 0.0)
        h = jnp.dot(xcat, fw_ref[...],
                    preferred_element_type=jnp.float32) + fb_ref[...]
        h = h * (bng_ref[...] * (1.0 / math.sqrt(1.0 + EPS))) + bnb_ref[...]
        o = jnp.dot(h, mw_ref[...],
                    preferred_element_type=jnp.float32) + mb_ref[...]
        m = jnp.max(o, axis=1, keepdims=True)
        s = o - m
        lse = jnp.log(jnp.sum(jnp.exp(s), axis=1, keepdims=True))
        o_ref[...] = s - lse


def _layer1_body(adj_ref, adj1_ref, adj2_ref, z_ref, b_ref, fw_ref, fb_ref,
                 bng_ref, bnb_ref, w2_ref, o_ref, acc_ref, *, nk):
    k = pl.program_id(1)

    @pl.when(k == 0)
    def _init():
        acc_ref[...] = jnp.zeros_like(acc_ref)

    z = z_ref[...]
    valid = jax.lax.broadcasted_iota(jnp.int32, z.shape, 0) < (N - k * KB)
    z = jnp.where(valid, z, 0.0)
    p0 = jnp.dot(adj_ref[...], z[:, 0:NHID],
                 preferred_element_type=jnp.float32)
    p1 = jnp.dot(adj1_ref[...], z[:, NHID:2 * NHID],
                 preferred_element_type=jnp.float32)
    p2 = jnp.dot(adj2_ref[...], z[:, 2 * NHID:3 * NHID],
                 preferred_element_type=jnp.float32)
    acc_ref[...] += jnp.concatenate([p0, p1, p2], axis=1)

    @pl.when(k == nk - 1)
    def _epilogue():
        xcat = jnp.maximum(acc_ref[...] + b_ref[...], 0.0)
        h = jnp.dot(xcat, fw_ref[...],
                    preferred_element_type=jnp.float32) + fb_ref[...]
        h = h * (bng_ref[...] * (1.0 / math.sqrt(1.0 + EPS))) + bnb_ref[...]
        o_ref[...] = jnp.dot(h, w2_ref[...],
                             preferred_element_type=jnp.float32)


def _adj_spec():
    return pl.BlockSpec((MB, KB), lambda i, k: (i, k))


def kernel(x, adj, adj1, adj2, gc11_w, gc11_b, gc12_w, gc12_b, gc13_w,
           gc13_b, gc21_w, gc21_b, gc22_w, gc22_b, gc23_w, gc23_b,
           fusion_w, fusion_b, fusion1_w, fusion1_b, mlp1_w, mlp1_b,
           bn1_g, bn1_b, bn2_g, bn2_b):
    f32 = jnp.float32
    w1cat = jnp.concatenate([gc11_w, gc12_w, gc13_w], axis=1)
    b1cat = jnp.concatenate([gc11_b, gc12_b, gc13_b])[None, :]
    w2cat = jnp.concatenate([gc21_w, gc22_w, gc23_w], axis=1)
    b2cat = jnp.concatenate([gc21_b, gc22_b, gc23_b])[None, :]

    zcat = pl.pallas_call(
        _zcat_body,
        grid=(10,),
        in_specs=[
            pl.BlockSpec((N // 10, NFEAT), lambda i: (i, 0)),
            pl.BlockSpec((NFEAT, 3 * NHID), lambda i: (0, 0)),
        ],
        out_specs=pl.BlockSpec((N // 10, 3 * NHID), lambda i: (i, 0)),
        out_shape=jax.ShapeDtypeStruct((N, 3 * NHID), f32),
    )(x, w1cat)

    small = lambda r, c: pl.BlockSpec((r, c), lambda i, k: (0, 0))
    zspec = pl.BlockSpec((KB, 3 * NHID), lambda i, k: (k, 0))

    z2cat = pl.pallas_call(
        functools.partial(_layer1_body, nk=NK),
        grid=(NM, NK),
        in_specs=[
            _adj_spec(), _adj_spec(), _adj_spec(), zspec,
            small(1, 3 * NHID),
            small(3 * NHID, NHID),
            small(1, NHID),
            small(1, NHID),
            small(1, NHID),
            small(NHID, 3 * NHID),
        ],
        out_specs=pl.BlockSpec((MB, 3 * NHID), lambda i, k: (i, 0)),
        out_shape=jax.ShapeDtypeStruct((N, 3 * NHID), f32),
        scratch_shapes=[pltpu.VMEM((MB, 3 * NHID), f32)],
        compiler_params=pltpu.CompilerParams(
            dimension_semantics=("parallel", "arbitrary")),
    )(adj, adj1, adj2, zcat, b1cat, fusion_w, fusion_b[None, :],
      bn1_g[None, :], bn1_b[None, :], w2cat)

    out = pl.pallas_call(
        functools.partial(_layer2_body, nk=NK),
        grid=(NM, NK),
        in_specs=[
            _adj_spec(), _adj_spec(), _adj_spec(), zspec,
            small(1, 3 * NHID),
            small(3 * NHID, NHID),
            small(1, NHID),
            small(1, NHID),
            small(1, NHID),
            small(NHID, NCLASS),
            small(1, NCLASS),
        ],
        out_specs=pl.BlockSpec((MB, NCLASS), lambda i, k: (i, 0)),
        out_shape=jax.ShapeDtypeStruct((N, NCLASS), f32),
        scratch_shapes=[pltpu.VMEM((MB, 3 * NHID), f32)],
        compiler_params=pltpu.CompilerParams(
            dimension_semantics=("parallel", "arbitrary")),
    )(adj, adj1, adj2, z2cat, b2cat, fusion1_w, fusion1_b[None, :],
      bn2_g[None, :], bn2_b[None, :], mlp1_w, mlp1_b[None, :])

    return out


# speedup vs baseline: 1.1520x; 1.1520x over previous
import functools
import math

import jax
import jax.numpy as jnp
from jax.experimental import pallas as pl
from jax.experimental.pallas import tpu as pltpu

N = 10000
NFEAT = 128
NHID = 64
NCLASS = 32
EPS = 1e-5

MB = 400
KB = 2048
NM = N // MB
NK = -(-N // KB)
NPAD = NK * KB


def _zcat_body(x_ref, w_ref, o_ref):
    o_ref[...] = jnp.dot(x_ref[...], w_ref[...],
                         preferred_element_type=jnp.float32)


def _layer2_body(adj_ref, adj1_ref, adj2_ref, z_ref, b_ref, fw_ref, fb_ref,
                 bng_ref, bnb_ref, mw_ref, mb_ref, o_ref, acc_ref, *, nk):
    k = pl.program_id(1)

    @pl.when(k == 0)
    def _init():
        acc_ref[...] = jnp.zeros_like(acc_ref)

    z = z_ref[pl.ds(k * KB, KB), :]
    p0 = jnp.dot(adj_ref[...], z[:, 0:NHID],
                 preferred_element_type=jnp.float32)
    p1 = jnp.dot(adj1_ref[...], z[:, NHID:2 * NHID],
                 preferred_element_type=jnp.float32)
    p2 = jnp.dot(adj2_ref[...], z[:, 2 * NHID:3 * NHID],
                 preferred_element_type=jnp.float32)
    acc_ref[...] += jnp.concatenate([p0, p1, p2], axis=1)

    @pl.when(k == nk - 1)
    def _epilogue():
        xcat = jnp.maximum(acc_ref[...] + b_ref[...], 0.0)
        h = jnp.dot(xcat, fw_ref[...],
                    preferred_element_type=jnp.float32) + fb_ref[...]
        h = h * (bng_ref[...] * (1.0 / math.sqrt(1.0 + EPS))) + bnb_ref[...]
        o = jnp.dot(h, mw_ref[...],
                    preferred_element_type=jnp.float32) + mb_ref[...]
        m = jnp.max(o, axis=1, keepdims=True)
        s = o - m
        lse = jnp.log(jnp.sum(jnp.exp(s), axis=1, keepdims=True))
        o_ref[...] = s - lse


def _layer1_body(adj_ref, adj1_ref, adj2_ref, z_ref, b_ref, fw_ref, fb_ref,
                 bng_ref, bnb_ref, w2_ref, o_ref, acc_ref, *, nk):
    k = pl.program_id(1)

    @pl.when(k == 0)
    def _init():
        acc_ref[...] = jnp.zeros_like(acc_ref)

    z = z_ref[pl.ds(k * KB, KB), :]
    p0 = jnp.dot(adj_ref[...], z[:, 0:NHID],
                 preferred_element_type=jnp.float32)
    p1 = jnp.dot(adj1_ref[...], z[:, NHID:2 * NHID],
                 preferred_element_type=jnp.float32)
    p2 = jnp.dot(adj2_ref[...], z[:, 2 * NHID:3 * NHID],
                 preferred_element_type=jnp.float32)
    acc_ref[...] += jnp.concatenate([p0, p1, p2], axis=1)

    @pl.when(k == nk - 1)
    def _epilogue():
        xcat = jnp.maximum(acc_ref[...] + b_ref[...], 0.0)
        h = jnp.dot(xcat, fw_ref[...],
                    preferred_element_type=jnp.float32) + fb_ref[...]
        h = h * (bng_ref[...] * (1.0 / math.sqrt(1.0 + EPS))) + bnb_ref[...]
        o_ref[...] = jnp.dot(h, w2_ref[...],
                             preferred_element_type=jnp.float32)


def _adj_spec():
    return pl.BlockSpec((MB, KB), lambda i, k: (i, k))


def kernel(x, adj, adj1, adj2, gc11_w, gc11_b, gc12_w, gc12_b, gc13_w,
           gc13_b, gc21_w, gc21_b, gc22_w, gc22_b, gc23_w, gc23_b,
           fusion_w, fusion_b, fusion1_w, fusion1_b, mlp1_w, mlp1_b,
           bn1_g, bn1_b, bn2_g, bn2_b):
    f32 = jnp.float32
    w1cat = jnp.concatenate([gc11_w, gc12_w, gc13_w], axis=1)
    b1cat = jnp.concatenate([gc11_b, gc12_b, gc13_b])[None, :]
    w2cat = jnp.concatenate([gc21_w, gc22_w, gc23_w], axis=1)
    b2cat = jnp.concatenate([gc21_b, gc22_b, gc23_b])[None, :]

    zcat = pl.pallas_call(
        _zcat_body,
        grid=(10,),
        in_specs=[
            pl.BlockSpec((N // 10, NFEAT), lambda i: (i, 0)),
            pl.BlockSpec((NFEAT, 3 * NHID), lambda i: (0, 0)),
        ],
        out_specs=pl.BlockSpec((N // 10, 3 * NHID), lambda i: (i, 0)),
        out_shape=jax.ShapeDtypeStruct((N, 3 * NHID), f32),
    )(x, w1cat)
    zcat = jnp.pad(zcat, ((0, NPAD - N), (0, 0)))

    small = lambda r, c: pl.BlockSpec((r, c), lambda i, k: (0, 0))
    zspec = pl.BlockSpec((NPAD, 3 * NHID), lambda i, k: (0, 0))

    z2cat = pl.pallas_call(
        functools.partial(_layer1_body, nk=NK),
        grid=(NM, NK),
        in_specs=[
            _adj_spec(), _adj_spec(), _adj_spec(), zspec,
            small(1, 3 * NHID),
            small(3 * NHID, NHID),
            small(1, NHID),
            small(1, NHID),
            small(1, NHID),
            small(NHID, 3 * NHID),
        ],
        out_specs=pl.BlockSpec((MB, 3 * NHID), lambda i, k: (i, 0)),
        out_shape=jax.ShapeDtypeStruct((N, 3 * NHID), f32),
        scratch_shapes=[pltpu.VMEM((MB, 3 * NHID), f32)],
        compiler_params=pltpu.CompilerParams(
            dimension_semantics=("parallel", "arbitrary")),
    )(adj, adj1, adj2, zcat, b1cat, fusion_w, fusion_b[None, :],
      bn1_g[None, :], bn1_b[None, :], w2cat)
    z2cat = jnp.pad(z2cat, ((0, NPAD - N), (0, 0)))

    out = pl.pallas_call(
        functools.partial(_layer2_body, nk=NK),
        grid=(NM, NK),
        in_specs=[
            _adj_spec(), _adj_spec(), _adj_spec(), zspec,
            small(1, 3 * NHID),
            small(3 * NHID, NHID),
            small(1, NHID),
            small(1, NHID),
            small(1, NHID),
            small(NHID, NCLASS),
            small(1, NCLASS),
        ],
        out_specs=pl.BlockSpec((MB, NCLASS), lambda i, k: (i, 0)),
        out_shape=jax.ShapeDtypeStruct((N, NCLASS), f32),
        scratch_shapes=[pltpu.VMEM((MB, 3 * NHID), f32)],
        compiler_params=pltpu.CompilerParams(
            dimension_semantics=("parallel", "arbitrary")),
    )(adj, adj1, adj2, z2cat, b2cat, fusion1_w, fusion1_b[None, :],
      bn2_g[None, :], bn2_b[None, :], mlp1_w, mlp1_b[None, :])

    return out


# speedup vs baseline: 1.1722x; 1.0175x over previous
import functools
import math

import jax
import jax.numpy as jnp
from jax.experimental import pallas as pl
from jax.experimental.pallas import tpu as pltpu

N = 10000
NFEAT = 128
NHID = 64
NCLASS = 32
EPS = 1e-5

MB = 400
KB = 2048
NM = N // MB
NK = -(-N // KB)
NPAD = NK * KB


def _zcat_body(x_ref, w_ref, o_ref):
    o_ref[...] = jnp.dot(x_ref[...], w_ref[...],
                         preferred_element_type=jnp.float32)


def _layer2_body(adj_ref, adj1_ref, adj2_ref, z_ref, b_ref, fw_ref, fb_ref,
                 bng_ref, bnb_ref, mw_ref, mb_ref, o_ref, acc_ref, *, nk):
    k = pl.program_id(1)

    @pl.when(k == 0)
    def _init():
        acc_ref[...] = jnp.zeros_like(acc_ref)

    z = z_ref[pl.ds(k * KB, KB), :]
    valid = jax.lax.broadcasted_iota(jnp.int32, z.shape, 0) < (N - k * KB)
    z = jnp.where(valid, z, 0.0)
    p0 = jnp.dot(adj_ref[...], z[:, 0:NHID],
                 preferred_element_type=jnp.float32)
    p1 = jnp.dot(adj1_ref[...], z[:, NHID:2 * NHID],
                 preferred_element_type=jnp.float32)
    p2 = jnp.dot(adj2_ref[...], z[:, 2 * NHID:3 * NHID],
                 preferred_element_type=jnp.float32)
    acc_ref[...] += jnp.concatenate([p0, p1, p2], axis=1)

    @pl.when(k == nk - 1)
    def _epilogue():
        xcat = jnp.maximum(acc_ref[...] + b_ref[...], 0.0)
        h = jnp.dot(xcat, fw_ref[...],
                    preferred_element_type=jnp.float32) + fb_ref[...]
        h = h * (bng_ref[...] * (1.0 / math.sqrt(1.0 + EPS))) + bnb_ref[...]
        o = jnp.dot(h, mw_ref[...],
                    preferred_element_type=jnp.float32) + mb_ref[...]
        m = jnp.max(o, axis=1, keepdims=True)
        s = o - m
        lse = jnp.log(jnp.sum(jnp.exp(s), axis=1, keepdims=True))
        o_ref[...] = s - lse


def _layer1_body(adj_ref, adj1_ref, adj2_ref, z_ref, b_ref, fw_ref, fb_ref,
                 bng_ref, bnb_ref, w2_ref, o_ref, acc_ref, *, nk):
    k = pl.program_id(1)

    @pl.when(k == 0)
    def _init():
        acc_ref[...] = jnp.zeros_like(acc_ref)

    z = z_ref[pl.ds(k * KB, KB), :]
    valid = jax.lax.broadcasted_iota(jnp.int32, z.shape, 0) < (N - k * KB)
    z = jnp.where(valid, z, 0.0)
    p0 = jnp.dot(adj_ref[...], z[:, 0:NHID],
                 preferred_element_type=jnp.float32)
    p1 = jnp.dot(adj1_ref[...], z[:, NHID:2 * NHID],
                 preferred_element_type=jnp.float32)
    p2 = jnp.dot(adj2_ref[...], z[:, 2 * NHID:3 * NHID],
                 preferred_element_type=jnp.float32)
    acc_ref[...] += jnp.concatenate([p0, p1, p2], axis=1)

    @pl.when(k == nk - 1)
    def _epilogue():
        xcat = jnp.maximum(acc_ref[...] + b_ref[...], 0.0)
        h = jnp.dot(xcat, fw_ref[...],
                    preferred_element_type=jnp.float32) + fb_ref[...]
        h = h * (bng_ref[...] * (1.0 / math.sqrt(1.0 + EPS))) + bnb_ref[...]
        o_ref[...] = jnp.dot(h, w2_ref[...],
                             preferred_element_type=jnp.float32)


def _adj_spec():
    return pl.BlockSpec((MB, KB), lambda i, k: (i, k))


def kernel(x, adj, adj1, adj2, gc11_w, gc11_b, gc12_w, gc12_b, gc13_w,
           gc13_b, gc21_w, gc21_b, gc22_w, gc22_b, gc23_w, gc23_b,
           fusion_w, fusion_b, fusion1_w, fusion1_b, mlp1_w, mlp1_b,
           bn1_g, bn1_b, bn2_g, bn2_b):
    f32 = jnp.float32
    w1cat = jnp.concatenate([gc11_w, gc12_w, gc13_w], axis=1)
    b1cat = jnp.concatenate([gc11_b, gc12_b, gc13_b])[None, :]
    w2cat = jnp.concatenate([gc21_w, gc22_w, gc23_w], axis=1)
    b2cat = jnp.concatenate([gc21_b, gc22_b, gc23_b])[None, :]

    zcat = pl.pallas_call(
        _zcat_body,
        grid=(10,),
        in_specs=[
            pl.BlockSpec((NPAD // 10, NFEAT), lambda i: (i, 0)),
            pl.BlockSpec((NFEAT, 3 * NHID), lambda i: (0, 0)),
        ],
        out_specs=pl.BlockSpec((NPAD // 10, 3 * NHID), lambda i: (i, 0)),
        out_shape=jax.ShapeDtypeStruct((NPAD, 3 * NHID), f32),
    )(x, w1cat)

    small = lambda r, c: pl.BlockSpec((r, c), lambda i, k: (0, 0))
    zspec = pl.BlockSpec((NPAD, 3 * NHID), lambda i, k: (0, 0))

    z2cat = pl.pallas_call(
        functools.partial(_layer1_body, nk=NK),
        grid=(NM, NK),
        in_specs=[
            _adj_spec(), _adj_spec(), _adj_spec(), zspec,
            small(1, 3 * NHID),
            small(3 * NHID, NHID),
            small(1, NHID),
            small(1, NHID),
            small(1, NHID),
            small(NHID, 3 * NHID),
        ],
        out_specs=pl.BlockSpec((MB, 3 * NHID), lambda i, k: (i, 0)),
        out_shape=jax.ShapeDtypeStruct((NPAD, 3 * NHID), f32),
        scratch_shapes=[pltpu.VMEM((MB, 3 * NHID), f32)],
        compiler_params=pltpu.CompilerParams(
            dimension_semantics=("parallel", "arbitrary")),
    )(adj, adj1, adj2, zcat, b1cat, fusion_w, fusion_b[None, :],
      bn1_g[None, :], bn1_b[None, :], w2cat)

    out = pl.pallas_call(
        functools.partial(_layer2_body, nk=NK),
        grid=(NM, NK),
        in_specs=[
            _adj_spec(), _adj_spec(), _adj_spec(), zspec,
            small(1, 3 * NHID),
            small(3 * NHID, NHID),
            small(1, NHID),
            small(1, NHID),
            small(1, NHID),
            small(NHID, NCLASS),
            small(1, NCLASS),
        ],
        out_specs=pl.BlockSpec((MB, NCLASS), lambda i, k: (i, 0)),
        out_shape=jax.ShapeDtypeStruct((N, NCLASS), f32),
        scratch_shapes=[pltpu.VMEM((MB, 3 * NHID), f32)],
        compiler_params=pltpu.CompilerParams(
            dimension_semantics=("parallel", "arbitrary")),
    )(adj, adj1, adj2, z2cat, b2cat, fusion1_w, fusion1_b[None, :],
      bn2_g[None, :], bn2_b[None, :], mlp1_w, mlp1_b[None, :])

    return out


# speedup vs baseline: 1.1890x; 1.0144x over previous
import functools
import math

import jax
import jax.numpy as jnp
from jax.experimental import pallas as pl
from jax.experimental.pallas import tpu as pltpu

N = 10000
NFEAT = 128
NHID = 64
NCLASS = 32
EPS = 1e-5

MB = 1000
KB = 1024
NM = N // MB
NK = -(-N // KB)
NPAD = NK * KB


def _zcat_body(x_ref, w_ref, o_ref):
    o_ref[...] = jnp.dot(x_ref[...], w_ref[...],
                         preferred_element_type=jnp.float32)


def _layer2_body(adj_ref, adj1_ref, adj2_ref, z_ref, b_ref, fw_ref, fb_ref,
                 bng_ref, bnb_ref, mw_ref, mb_ref, o_ref, acc_ref, *, nk):
    k = pl.program_id(1)

    @pl.when(k == 0)
    def _init():
        acc_ref[...] = jnp.zeros_like(acc_ref)

    z = z_ref[pl.ds(k * KB, KB), :]
    valid = jax.lax.broadcasted_iota(jnp.int32, z.shape, 0) < (N - k * KB)
    z = jnp.where(valid, z, 0.0)
    p0 = jnp.dot(adj_ref[...], z[:, 0:NHID],
                 preferred_element_type=jnp.float32)
    p1 = jnp.dot(adj1_ref[...], z[:, NHID:2 * NHID],
                 preferred_element_type=jnp.float32)
    p2 = jnp.dot(adj2_ref[...], z[:, 2 * NHID:3 * NHID],
                 preferred_element_type=jnp.float32)
    acc_ref[...] += jnp.concatenate([p0, p1, p2], axis=1)

    @pl.when(k == nk - 1)
    def _epilogue():
        xcat = jnp.maximum(acc_ref[...] + b_ref[...], 0.0)
        h = jnp.dot(xcat, fw_ref[...],
                    preferred_element_type=jnp.float32) + fb_ref[...]
        h = h * (bng_ref[...] * (1.0 / math.sqrt(1.0 + EPS))) + bnb_ref[...]
        o = jnp.dot(h, mw_ref[...],
                    preferred_element_type=jnp.float32) + mb_ref[...]
        m = jnp.max(o, axis=1, keepdims=True)
        s = o - m
        lse = jnp.log(jnp.sum(jnp.exp(s), axis=1, keepdims=True))
        o_ref[...] = s - lse


def _layer1_body(adj_ref, adj1_ref, adj2_ref, z_ref, b_ref, fw_ref, fb_ref,
                 bng_ref, bnb_ref, w2_ref, o_ref, acc_ref, *, nk):
    k = pl.program_id(1)

    @pl.when(k == 0)
    def _init():
        acc_ref[...] = jnp.zeros_like(acc_ref)

    z = z_ref[pl.ds(k * KB, KB), :]
    valid = jax.lax.broadcasted_iota(jnp.int32, z.shape, 0) < (N - k * KB)
    z = jnp.where(valid, z, 0.0)
    p0 = jnp.dot(adj_ref[...], z[:, 0:NHID],
                 preferred_element_type=jnp.float32)
    p1 = jnp.dot(adj1_ref[...], z[:, NHID:2 * NHID],
                 preferred_element_type=jnp.float32)
    p2 = jnp.dot(adj2_ref[...], z[:, 2 * NHID:3 * NHID],
                 preferred_element_type=jnp.float32)
    acc_ref[...] += jnp.concatenate([p0, p1, p2], axis=1)

    @pl.when(k == nk - 1)
    def _epilogue():
        xcat = jnp.maximum(acc_ref[...] + b_ref[...], 0.0)
        h = jnp.dot(xcat, fw_ref[...],
                    preferred_element_type=jnp.float32) + fb_ref[...]
        h = h * (bng_ref[...] * (1.0 / math.sqrt(1.0 + EPS))) + bnb_ref[...]
        o_ref[...] = jnp.dot(h, w2_ref[...],
                             preferred_element_type=jnp.float32)


def _adj_spec():
    return pl.BlockSpec((MB, KB), lambda i, k: (i, k))


def kernel(x, adj, adj1, adj2, gc11_w, gc11_b, gc12_w, gc12_b, gc13_w,
           gc13_b, gc21_w, gc21_b, gc22_w, gc22_b, gc23_w, gc23_b,
           fusion_w, fusion_b, fusion1_w, fusion1_b, mlp1_w, mlp1_b,
           bn1_g, bn1_b, bn2_g, bn2_b):
    f32 = jnp.float32
    w1cat = jnp.concatenate([gc11_w, gc12_w, gc13_w], axis=1)
    b1cat = jnp.concatenate([gc11_b, gc12_b, gc13_b])[None, :]
    w2cat = jnp.concatenate([gc21_w, gc22_w, gc23_w], axis=1)
    b2cat = jnp.concatenate([gc21_b, gc22_b, gc23_b])[None, :]

    zcat = pl.pallas_call(
        _zcat_body,
        grid=(10,),
        in_specs=[
            pl.BlockSpec((NPAD // 10, NFEAT), lambda i: (i, 0)),
            pl.BlockSpec((NFEAT, 3 * NHID), lambda i: (0, 0)),
        ],
        out_specs=pl.BlockSpec((NPAD // 10, 3 * NHID), lambda i: (i, 0)),
        out_shape=jax.ShapeDtypeStruct((NPAD, 3 * NHID), f32),
    )(x, w1cat)

    small = lambda r, c: pl.BlockSpec((r, c), lambda i, k: (0, 0))
    zspec = pl.BlockSpec((NPAD, 3 * NHID), lambda i, k: (0, 0))

    z2cat = pl.pallas_call(
        functools.partial(_layer1_body, nk=NK),
        grid=(NM, NK),
        in_specs=[
            _adj_spec(), _adj_spec(), _adj_spec(), zspec,
            small(1, 3 * NHID),
            small(3 * NHID, NHID),
            small(1, NHID),
            small(1, NHID),
            small(1, NHID),
            small(NHID, 3 * NHID),
        ],
        out_specs=pl.BlockSpec((MB, 3 * NHID), lambda i, k: (i, 0)),
        out_shape=jax.ShapeDtypeStruct((NPAD, 3 * NHID), f32),
        scratch_shapes=[pltpu.VMEM((MB, 3 * NHID), f32)],
        compiler_params=pltpu.CompilerParams(
            dimension_semantics=("parallel", "arbitrary")),
    )(adj, adj1, adj2, zcat, b1cat, fusion_w, fusion_b[None, :],
      bn1_g[None, :], bn1_b[None, :], w2cat)

    out = pl.pallas_call(
        functools.partial(_layer2_body, nk=NK),
        grid=(NM, NK),
        in_specs=[
            _adj_spec(), _adj_spec(), _adj_spec(), zspec,
            small(1, 3 * NHID),
            small(3 * NHID, NHID),
            small(1, NHID),
            small(1, NHID),
            small(1, NHID),
            small(NHID, NCLASS),
            small(1, NCLASS),
        ],
        out_specs=pl.BlockSpec((MB, NCLASS), lambda i, k: (i, 0)),
        out_shape=jax.ShapeDtypeStruct((N, NCLASS), f32),
        scratch_shapes=[pltpu.VMEM((MB, 3 * NHID), f32)],
        compiler_params=pltpu.CompilerParams(
            dimension_semantics=("parallel", "arbitrary")),
    )(adj, adj1, adj2, z2cat, b2cat, fusion1_w, fusion1_b[None, :],
      bn2_g[None, :], bn2_b[None, :], mlp1_w, mlp1_b[None, :])

    return out
